# baseline (speedup 1.0000x reference)
import functools

import jax
import jax.numpy as jnp
from jax import lax
from jax.experimental import pallas as pl
from jax.experimental.pallas import tpu as pltpu
from jax.experimental.pallas import tpu_sc as plsc

VOCAB = 1000000
EMBED = 64
BATCH = 4096
HIST = 200
TOTAL = BATCH * HIST

NC, NS = 2, 16
NW = NC * NS

TBLK = 256
NBLK = VOCAB // TBLK
TAILC = VOCAB - NBLK * TBLK

BPW = TOTAL // NW
BATW = BATCH // NW
CHUNK = HIST
NCHUNKS = BPW // CHUNK
HALF = BPW // 2
SPLITS = ((0, 104), (104, 96))
SCALE = 8.0

_mesh = plsc.VectorSubcoreMesh(core_axis_name="c", subcore_axis_name="s")
_params = pltpu.CompilerParams(
    use_tc_tiling_on_sc=True, needs_layout_passes=False)


@functools.partial(
    pl.kernel,
    out_type=jax.ShapeDtypeStruct((VOCAB, 128), jnp.float32),
    mesh=_mesh,
    scratch_types=[
        pltpu.VMEM((2, EMBED, TBLK), jnp.float32),
        pltpu.VMEM((2, TBLK, 128), jnp.float32),
        pltpu.VMEM((TAILC, EMBED), jnp.float32),
        pltpu.VMEM((TAILC, 128), jnp.float32),
        pltpu.SemaphoreType.DMA,
        pltpu.SemaphoreType.DMA,
    ],
    compiler_params=_params,
)
def _transpose(embt_hbm, tail_hbm, p_hbm, in_v, out_v, tin_v, tout_v,
               sem0, sem1):
    wid = lax.axis_index("s") * NC + lax.axis_index("c")
    sems = (sem0, sem1)

    nblk_w = NBLK // NW + 1

    def fire(buf, blk):
        c0 = pl.multiple_of(blk * TBLK, TBLK)
        pltpu.async_copy(
            embt_hbm.at[:, pl.ds(c0, TBLK)], in_v.at[buf], sems[buf])

    def drain(buf):
        pltpu.make_async_copy(
            embt_hbm.at[:, pl.ds(0, TBLK)], in_v.at[buf], sems[buf]).wait()

    def transpose_block(src, dst, nrows):
        @plsc.parallel_loop(0, nrows, step=1, unroll=4)
        def _row(r):
            for j in range(EMBED // 16):
                v = plsc.load_gather(
                    src, [j * 16 + lax.iota(jnp.int32, 16),
                          jnp.full((16,), r, jnp.int32)])
                dst[r, pl.ds(j * 16, 16)] = v * SCALE

    fire(0, wid)
    fire(1, wid + NW)

    @pl.loop(0, nblk_w, step=2)
    def _blocks(k):
        for b in (0, 1):
            slot = k + b
            blk = wid + slot * NW

            @pl.when(blk < NBLK)
            def _():
                drain(b)
                transpose_block(in_v.at[b], out_v.at[b], TBLK)
                r0 = pl.multiple_of(blk * TBLK, TBLK)
                pltpu.sync_copy(out_v.at[b], p_hbm.at[pl.ds(r0, TBLK)])
                nxt = blk + 2 * NW

                @pl.when(nxt < NBLK)
                def _():
                    fire(b, nxt)

    @pl.when(wid == 0)
    def _tail():
        pltpu.sync_copy(tail_hbm, tin_v)

        @plsc.parallel_loop(0, TAILC, step=1, unroll=4)
        def _row(r):
            for j in range(EMBED // 16):
                tout_v[r, pl.ds(j * 16, 16)] = (
                    tin_v[r, pl.ds(j * 16, 16)] * SCALE)

        r0 = pl.multiple_of(NBLK * TBLK, 8)
        pltpu.sync_copy(tout_v, p_hbm.at[pl.ds(r0, TAILC)])


@functools.partial(
    pl.kernel,
    out_type=jax.ShapeDtypeStruct((BATCH, HIST, EMBED), jnp.float32),
    mesh=_mesh,
    scratch_types=[
        pltpu.VMEM((BPW,), jnp.int32),
        pltpu.VMEM((2, CHUNK, 128), jnp.float32),
        pltpu.VMEM((2, CHUNK, EMBED), jnp.float32),
        pltpu.SemaphoreType.DMA,
        pltpu.SemaphoreType.DMA,
    ],
    compiler_params=_params,
)
def _gather_scale(x_hbm, p_hbm, out_hbm, idx_v, rows_v, comp_v, sem0, sem1):
    wid = lax.axis_index("s") * NC + lax.axis_index("c")
    base = wid * BPW
    bat0 = wid * BATW
    sems = (sem0, sem1)

    def fire(buf, chunk):
        for (o, n) in SPLITS:
            ioff = pl.multiple_of(chunk * CHUNK, 8) + o
            pltpu.async_copy(
                p_hbm.at[idx_v.at[pl.ds(ioff, n)]],
                rows_v.at[buf, pl.ds(o, n)],
                sems[buf],
            )

    def drain(buf):
        for (o, n) in SPLITS:
            pltpu.make_async_copy(
                p_hbm.at[idx_v.at[pl.ds(o, n)]],
                rows_v.at[buf, pl.ds(o, n)],
                sems[buf],
            ).wait()

    pltpu.sync_copy(
        x_hbm.at[pl.ds(pl.multiple_of(base, BPW), BPW)], idx_v)
    fire(0, 0)

    @pl.loop(0, NCHUNKS, step=2)
    def _steps(ci):
        for b in (0, 1):
            cur = ci + b

            @pl.when(cur + 1 < NCHUNKS)
            def _():
                fire(1 - b, cur + 1)

            drain(b)

            rv = rows_v.at[b]
            cv = comp_v.at[b]

            @plsc.parallel_loop(0, CHUNK, step=1, unroll=4)
            def _select(r):
                for j in range(EMBED // 16):
                    cv[r, pl.ds(j * 16, 16)] = rv[r, pl.ds(j * 16, 16)]

            pltpu.sync_copy(cv, out_hbm.at[bat0 + cur])


def kernel(x, embedding):
    p = _transpose(embedding.T, embedding[NBLK * TBLK:, :])
    return _gather_scale(x.reshape(TOTAL), p)


# speedup vs baseline: 1.2409x; 1.2409x over previous
import functools

import jax
import jax.numpy as jnp
from jax import lax
from jax.experimental import pallas as pl
from jax.experimental.pallas import tpu as pltpu
from jax.experimental.pallas import tpu_sc as plsc

VOCAB = 1000000
EMBED = 64
BATCH = 4096
HIST = 200
TOTAL = BATCH * HIST

NC, NS = 2, 16
NW = NC * NS
BPW = TOTAL // NW
BATW = BATCH // NW
IDXW = 80
NBAT = 2
CHUNK = NBAT * HIST
GATHERS = CHUNK // IDXW
NCHUNKS = BPW // CHUNK
SCALE = 8.0

_mesh = plsc.VectorSubcoreMesh(core_axis_name="c", subcore_axis_name="s")


@functools.partial(
    pl.kernel,
    out_type=jax.ShapeDtypeStruct((BATCH, HIST, EMBED), jnp.float32),
    mesh=_mesh,
    scratch_types=[
        pltpu.VMEM((BPW,), jnp.int32),
        pltpu.VMEM((2, CHUNK, EMBED), jnp.float32),
        pltpu.SemaphoreType.DMA,
        pltpu.SemaphoreType.DMA,
    ],
    compiler_params=pltpu.CompilerParams(use_tc_tiling_on_sc=False),
)
def _gather_scale(x_hbm, emb_hbm, out_hbm, idx_v, rows_v, sem0, sem1):
    wid = lax.axis_index("s") * NC + lax.axis_index("c")
    base = wid * BPW
    bat0 = wid * BATW
    sems = (sem0, sem1)

    pltpu.sync_copy(x_hbm.at[pl.ds(pl.multiple_of(base, BPW), BPW)], idx_v)

    def fire(buf, chunk):
        for g in range(GATHERS):
            off = pl.multiple_of(chunk * CHUNK + g * IDXW, IDXW)
            pltpu.async_copy(
                emb_hbm.at[idx_v.at[pl.ds(off, IDXW)]],
                rows_v.at[buf, pl.ds(g * IDXW, IDXW)],
                sems[buf],
            )

    def drain(buf):
        for g in range(GATHERS):
            pltpu.make_async_copy(
                emb_hbm.at[idx_v.at[pl.ds(g * IDXW, IDXW)]],
                rows_v.at[buf, pl.ds(g * IDXW, IDXW)],
                sems[buf],
            ).wait()

    fire(0, 0)

    @pl.loop(0, NCHUNKS, step=2)
    def _steps(ci):
        for b in (0, 1):
            cur = ci + b

            @pl.when(cur + 1 < NCHUNKS)
            def _():
                fire(1 - b, cur + 1)

            drain(b)

            @plsc.parallel_loop(0, CHUNK, step=1, unroll=8)
            def _scale(r):
                for j in range(EMBED // 16):
                    sl = pl.ds(j * 16, 16)
                    rows_v[b, r, sl] = rows_v[b, r, sl] * SCALE

            for k in range(NBAT):
                pltpu.sync_copy(
                    rows_v.at[b, pl.ds(k * HIST, HIST)],
                    out_hbm.at[bat0 + cur * NBAT + k],
                )


def kernel(x, embedding):
    x2 = x.reshape(TOTAL)
    return _gather_scale(x2, embedding)
